# baseline (speedup 1.0000x reference)
import functools

import jax
import jax.numpy as jnp
from jax import lax
from jax.experimental import pallas as pl
from jax.experimental.pallas import tpu as pltpu
from jax.experimental.pallas import tpu_sc as plsc

HQ = 16384
W = 32
D = 128
NW = 32
CQ = 256
NCHUNK = HQ // CQ
OUTC = W * D

_mesh = plsc.VectorSubcoreMesh(core_axis_name="c", subcore_axis_name="s")


@functools.partial(
    pl.kernel,
    mesh=_mesh,
    compiler_params=pltpu.CompilerParams(
        use_tc_tiling_on_sc=True, needs_layout_passes=False),
    out_type=jax.ShapeDtypeStruct((HQ, OUTC), jnp.float32),
    scratch_types=[
        pltpu.VMEM((CQ * W,), jnp.int32),
        pltpu.VMEM((CQ * W,), jnp.int32),
        pltpu.VMEM((CQ,), jnp.int32),
        pltpu.VMEM((CQ,), jnp.int32),
        pltpu.VMEM((CQ, D), jnp.float32),
        pltpu.VMEM((CQ, D), jnp.float32),
        pltpu.SemaphoreType.DMA,
        pltpu.SemaphoreType.DMA,
        pltpu.SemaphoreType.DMA,
        pltpu.SemaphoreType.DMA,
        pltpu.SemaphoreType.DMA,
        pltpu.SemaphoreType.DMA,
    ],
)
def _gather_kernel(x_hbm, idx_hbm, out_hbm,
                   sidx_0, sidx_1, col_0, col_1, rows_0, rows_1,
                   sem_s0, sem_s1, sem_g0, sem_g1, sem_o0, sem_o1):
    wid = lax.axis_index("s") * 2 + lax.axis_index("c")
    lanes = lax.iota(jnp.int32, 16)

    sidx_b = (sidx_0, sidx_1)
    col_b = (col_0, col_1)
    rows_b = (rows_0, rows_1)
    sem_sb = (sem_s0, sem_s1)
    sem_gb = (sem_g0, sem_g1)
    sem_ob = (sem_o0, sem_o1)

    def fire_stage(c, b):
        c = jnp.minimum(c, NCHUNK - 1)
        pltpu.async_copy(idx_hbm.at[pl.ds(c * CQ * W, CQ * W)], sidx_b[b],
                         sem_sb[b])

    def wait_stage(b):
        pltpu.make_async_copy(idx_hbm.at[pl.ds(0, CQ * W)], sidx_b[b],
                              sem_sb[b]).wait()

    def extract(b):
        sidx_v, col_v = sidx_b[b], col_b[b]
        for t in range(CQ // 16):
            pos = lanes * W + (512 * t) + wid
            col_v[pl.ds(16 * t, 16)] = plsc.load_gather(sidx_v, [pos])

    def fire_gather(b):
        col_v, rows_v = col_b[b], rows_b[b]
        for u in range(CQ // 128):
            pltpu.async_copy(
                x_hbm.at[col_v.at[pl.ds(u * 128, 128)]],
                rows_v.at[pl.ds(u * 128, 128)],
                sem_gb[b],
            )

    def wait_gather(b):
        pltpu.make_async_copy(
            x_hbm.at[pl.ds(0, CQ)], rows_b[b].at[pl.ds(0, CQ)],
            sem_gb[b]).wait()

    def fire_out(c, b):
        pltpu.async_copy(
            rows_b[b],
            out_hbm.at[pl.ds(c * CQ, CQ), pl.ds(wid * D, D)],
            sem_ob[b],
        )

    def drain_out(b):
        pltpu.make_async_copy(
            rows_b[b], out_hbm.at[pl.ds(0, CQ), pl.ds(0, D)],
            sem_ob[b]).wait()

    fire_stage(0, 0)
    wait_stage(0)
    extract(0)
    fire_stage(1, 1)
    fire_gather(0)
    wait_stage(1)
    extract(1)
    fire_stage(2, 0)
    fire_gather(1)
    wait_gather(0)
    fire_out(0, 0)

    def pair_body(s, carry):
        for b in range(2):
            c = 2 * s + b
            wait_stage(b)
            extract(b)
            fire_stage(c + 1, 1 - b)
            drain_out(b)
            fire_gather(b)
            wait_gather(1 - b)
            fire_out(c - 1, 1 - b)
        return carry

    lax.fori_loop(1, NCHUNK // 2, pair_body, 0)

    wait_gather(1)
    fire_out(NCHUNK - 1, 1)

    wait_stage(0)
    drain_out(0)
    drain_out(1)


def kernel(x, idx, dis, angle):
    idx32 = idx.astype(jnp.int32).reshape(-1)
    g = _gather_kernel(x, idx32)
    return jnp.concatenate([g, dis, angle], axis=1)


# speedup vs baseline: 1.6041x; 1.6041x over previous
import functools

import jax
import jax.numpy as jnp
from jax import lax
from jax.experimental import pallas as pl
from jax.experimental.pallas import tpu as pltpu
from jax.experimental.pallas import tpu_sc as plsc

HQ = 16384
W = 32
D = 128
NW = 32
CQ = 256
NCHUNK = HQ // CQ
OUTC = W * D + 2 * W

_mesh = plsc.VectorSubcoreMesh(core_axis_name="c", subcore_axis_name="s")


@functools.partial(
    pl.kernel,
    mesh=_mesh,
    compiler_params=pltpu.CompilerParams(
        use_tc_tiling_on_sc=True, needs_layout_passes=False),
    out_type=jax.ShapeDtypeStruct((HQ, OUTC), jnp.float32),
    scratch_types=[
        pltpu.VMEM((CQ * W,), jnp.int32),
        pltpu.VMEM((CQ * W,), jnp.int32),
        pltpu.VMEM((CQ,), jnp.int32),
        pltpu.VMEM((CQ,), jnp.int32),
        pltpu.VMEM((CQ, D), jnp.float32),
        pltpu.VMEM((CQ, D), jnp.float32),
        pltpu.SemaphoreType.DMA,
        pltpu.SemaphoreType.DMA,
        pltpu.SemaphoreType.DMA,
        pltpu.SemaphoreType.DMA,
        pltpu.SemaphoreType.DMA,
        pltpu.SemaphoreType.DMA,
    ],
)
def _gather_kernel(x_hbm, idx_hbm, out_hbm,
                   sidx_0, sidx_1, col_0, col_1, rows_0, rows_1,
                   sem_s0, sem_s1, sem_g0, sem_g1, sem_o0, sem_o1):
    wid = lax.axis_index("s") * 2 + lax.axis_index("c")
    lanes = lax.iota(jnp.int32, 16)

    sidx_b = (sidx_0, sidx_1)
    col_b = (col_0, col_1)
    rows_b = (rows_0, rows_1)
    sem_sb = (sem_s0, sem_s1)
    sem_gb = (sem_g0, sem_g1)
    sem_ob = (sem_o0, sem_o1)

    def fire_stage(c, b):
        c = jnp.minimum(c, NCHUNK - 1)
        pltpu.async_copy(idx_hbm.at[pl.ds(c * CQ * W, CQ * W)], sidx_b[b],
                         sem_sb[b])

    def wait_stage(b):
        pltpu.make_async_copy(idx_hbm.at[pl.ds(0, CQ * W)], sidx_b[b],
                              sem_sb[b]).wait()

    def extract(b):
        sidx_v, col_v = sidx_b[b], col_b[b]
        for t in range(CQ // 16):
            pos = lanes * W + (512 * t) + wid
            col_v[pl.ds(16 * t, 16)] = plsc.load_gather(sidx_v, [pos])

    def fire_gather(b):
        col_v, rows_v = col_b[b], rows_b[b]
        for u in range(CQ // 128):
            pltpu.async_copy(
                x_hbm.at[col_v.at[pl.ds(u * 128, 128)]],
                rows_v.at[pl.ds(u * 128, 128)],
                sem_gb[b],
            )

    def wait_gather(b):
        pltpu.make_async_copy(
            x_hbm.at[pl.ds(0, CQ)], rows_b[b].at[pl.ds(0, CQ)],
            sem_gb[b]).wait()

    def fire_out(c, b):
        pltpu.async_copy(
            rows_b[b],
            out_hbm.at[pl.ds(c * CQ, CQ), pl.ds(wid * D, D)],
            sem_ob[b],
        )

    def drain_out(b):
        pltpu.make_async_copy(
            rows_b[b], out_hbm.at[pl.ds(0, CQ), pl.ds(0, D)],
            sem_ob[b]).wait()

    fire_stage(0, 0)
    wait_stage(0)
    extract(0)
    fire_stage(1, 1)
    fire_gather(0)
    wait_stage(1)
    extract(1)
    fire_stage(2, 0)
    fire_gather(1)
    wait_gather(0)
    fire_out(0, 0)

    def pair_body(s, carry):
        for b in range(2):
            c = 2 * s + b
            wait_stage(b)
            extract(b)
            fire_stage(c + 1, 1 - b)
            drain_out(b)
            fire_gather(b)
            wait_gather(1 - b)
            fire_out(c - 1, 1 - b)
        return carry

    lax.fori_loop(1, NCHUNK // 2, pair_body, 0)

    wait_gather(1)
    fire_out(NCHUNK - 1, 1)

    wait_stage(0)
    drain_out(0)
    drain_out(1)


def kernel(x, idx, dis, angle):
    idx32 = idx.astype(jnp.int32).reshape(-1)
    out = _gather_kernel(x, idx32)
    da = jnp.concatenate([dis, angle], axis=1)
    return lax.dynamic_update_slice(out, da, (0, W * D))


# speedup vs baseline: 1.7652x; 1.1004x over previous
import functools

import jax
import jax.numpy as jnp
from jax import lax
from jax.experimental import pallas as pl
from jax.experimental.pallas import tpu as pltpu
from jax.experimental.pallas import tpu_sc as plsc

HQ = 16384
W = 32
D = 128
NW = 32
CQ = 256
NCHUNK = HQ // CQ
OUTC = W * D + 2 * W

_mesh = plsc.VectorSubcoreMesh(core_axis_name="c", subcore_axis_name="s")


@functools.partial(
    pl.kernel,
    mesh=_mesh,
    compiler_params=pltpu.CompilerParams(
        use_tc_tiling_on_sc=True, needs_layout_passes=False),
    out_type=jax.ShapeDtypeStruct((HQ, OUTC), jnp.float32),
    scratch_types=[
        pltpu.VMEM((HQ,), jnp.int32),
        pltpu.VMEM((CQ, D), jnp.float32),
        pltpu.VMEM((CQ, D), jnp.float32),
        pltpu.SemaphoreType.DMA,
        pltpu.SemaphoreType.DMA,
        pltpu.SemaphoreType.DMA,
        pltpu.SemaphoreType.DMA,
    ],
)
def _gather_kernel(x_hbm, idxt_hbm, out_hbm,
                   col_all, rows_0, rows_1,
                   sem_g0, sem_g1, sem_o0, sem_o1):
    wid = lax.axis_index("s") * 2 + lax.axis_index("c")

    rows_b = (rows_0, rows_1)
    sem_gb = (sem_g0, sem_g1)
    sem_ob = (sem_o0, sem_o1)

    pltpu.sync_copy(idxt_hbm.at[pl.ds(wid * HQ, HQ)], col_all)

    def fire_gather(c, b):
        rows_v = rows_b[b]
        for u in range(CQ // 128):
            pltpu.async_copy(
                x_hbm.at[col_all.at[pl.ds(c * CQ + u * 128, 128)]],
                rows_v.at[pl.ds(u * 128, 128)],
                sem_gb[b],
            )

    def wait_gather(b):
        pltpu.make_async_copy(
            x_hbm.at[pl.ds(0, CQ)], rows_b[b].at[pl.ds(0, CQ)],
            sem_gb[b]).wait()

    def fire_out(c, b):
        pltpu.async_copy(
            rows_b[b],
            out_hbm.at[pl.ds(c * CQ, CQ), pl.ds(wid * D, D)],
            sem_ob[b],
        )

    def drain_out(b):
        pltpu.make_async_copy(
            rows_b[b], out_hbm.at[pl.ds(0, CQ), pl.ds(0, D)],
            sem_ob[b]).wait()

    fire_gather(0, 0)
    fire_gather(1, 1)
    wait_gather(0)
    fire_out(0, 0)

    def pair_body(s, carry):
        for b in range(2):
            c = 2 * s + b
            drain_out(b)
            fire_gather(c, b)
            wait_gather(1 - b)
            fire_out(c - 1, 1 - b)
        return carry

    lax.fori_loop(1, NCHUNK // 2, pair_body, 0)

    wait_gather(1)
    fire_out(NCHUNK - 1, 1)
    drain_out(0)
    drain_out(1)


def kernel(x, idx, dis, angle):
    idxt = idx.astype(jnp.int32).T.reshape(-1)
    out = _gather_kernel(x, idxt)
    da = jnp.concatenate([dis, angle], axis=1)
    return lax.dynamic_update_slice(out, da, (0, W * D))
